# baseline (speedup 1.0000x reference)
import jax
import jax.numpy as jnp
from jax.experimental import pallas as pl
from jax.experimental.pallas import tpu as pltpu

_T = 16
_N = 1000000
_V_TH = 1.0

_C = 65536
_NBLK = (_N + _C - 1) // _C


def _body(t_ref, x_ref, v_ref, s_ref, o_ref):
    t = t_ref[0]
    vnew = v_ref[0, :] + x_ref[0, 0, :]
    fire = vnew >= _V_TH
    o_ref[...] = s_ref[...]
    o_ref[pl.ds(t, 1), :] = jnp.where(fire[None, :], jnp.float32(1.0),
                                      s_ref[pl.ds(t, 1), :])


def kernel(t, x, v, s):
    v2 = v.reshape(1, _N)
    s2 = s.reshape(_T, _N)
    t_arr = jnp.asarray(t, jnp.int32).reshape(1)

    grid_spec = pltpu.PrefetchScalarGridSpec(
        num_scalar_prefetch=1,
        grid=(_NBLK,),
        in_specs=[
            pl.BlockSpec((1, 1, _C), lambda j, t_ref: (t_ref[0], 0, j)),
            pl.BlockSpec((1, _C), lambda j, t_ref: (0, j)),
            pl.BlockSpec((_T, _C), lambda j, t_ref: (0, j)),
        ],
        out_specs=pl.BlockSpec((_T, _C), lambda j, t_ref: (0, j)),
    )
    out = pl.pallas_call(
        _body,
        grid_spec=grid_spec,
        out_shape=jax.ShapeDtypeStruct((_T, _N), jnp.float32),
    )(t_arr, x, v2, s2)
    return out.reshape(_T, 1, _N)


# speedup vs baseline: 6.7355x; 6.7355x over previous
import jax
import jax.numpy as jnp
from jax.experimental import pallas as pl
from jax.experimental.pallas import tpu as pltpu

_T = 16
_N = 1000000
_V_TH = 1.0

_C = 65536
_NBLK = (_N + _C - 1) // _C


def _body(t_ref, x_ref, v_ref, s_ref, o_ref):
    t = t_ref[0]
    vnew = v_ref[...] + x_ref[0, 0, :]
    fire = vnew >= _V_TH
    o_ref[...] = s_ref[...]
    o_ref[pl.ds(t, 1), 0, :] = jnp.where(fire[None, :], jnp.float32(1.0),
                                         s_ref[pl.ds(t, 1), 0, :])


def kernel(t, x, v, s):
    t_arr = jnp.asarray(t, jnp.int32).reshape(1)

    grid_spec = pltpu.PrefetchScalarGridSpec(
        num_scalar_prefetch=1,
        grid=(_NBLK,),
        in_specs=[
            pl.BlockSpec((1, 1, _C), lambda j, t_ref: (t_ref[0], 0, j)),
            pl.BlockSpec((_C,), lambda j, t_ref: (j,)),
            pl.BlockSpec((_T, 1, _C), lambda j, t_ref: (0, 0, j)),
        ],
        out_specs=pl.BlockSpec((_T, 1, _C), lambda j, t_ref: (0, 0, j)),
    )
    return pl.pallas_call(
        _body,
        grid_spec=grid_spec,
        out_shape=jax.ShapeDtypeStruct((_T, 1, _N), jnp.float32),
    )(t_arr, x, v, s)


# speedup vs baseline: 6.8776x; 1.0211x over previous
import jax
import jax.numpy as jnp
from jax.experimental import pallas as pl
from jax.experimental.pallas import tpu as pltpu

_T = 16
_N = 1000000
_V_TH = 1.0

_C = 131072
_NBLK = (_N + _C - 1) // _C


def _body(t_ref, x_ref, v_ref, s_ref, o_ref):
    t = t_ref[0]
    vnew = v_ref[...] + x_ref[0, 0, :]
    fire = vnew >= _V_TH
    o_ref[...] = s_ref[...]
    o_ref[pl.ds(t, 1), 0, :] = jnp.where(fire[None, :], jnp.float32(1.0),
                                         s_ref[pl.ds(t, 1), 0, :])


def kernel(t, x, v, s):
    t_arr = jnp.asarray(t, jnp.int32).reshape(1)

    grid_spec = pltpu.PrefetchScalarGridSpec(
        num_scalar_prefetch=1,
        grid=(_NBLK,),
        in_specs=[
            pl.BlockSpec((1, 1, _C), lambda j, t_ref: (t_ref[0], 0, j)),
            pl.BlockSpec((_C,), lambda j, t_ref: (j,)),
            pl.BlockSpec((_T, 1, _C), lambda j, t_ref: (0, 0, j)),
        ],
        out_specs=pl.BlockSpec((_T, 1, _C), lambda j, t_ref: (0, 0, j)),
    )
    return pl.pallas_call(
        _body,
        grid_spec=grid_spec,
        out_shape=jax.ShapeDtypeStruct((_T, 1, _N), jnp.float32),
    )(t_arr, x, v, s)
